# baseline (speedup 1.0000x reference)
import functools

import jax
import jax.numpy as jnp
from jax import lax
from jax.experimental import pallas as pl
from jax.experimental.pallas import tpu as pltpu
from jax.experimental.pallas import tpu_sc as plsc

EPS = 1e-12
NC, NS, LANES = 2, 16, 16
NW = NC * NS


def _rsqrt_vec(v):
    bits = plsc.bitcast(v, jnp.int32)
    magic = jnp.full((LANES,), 0x5F3759DF, dtype=jnp.int32)
    y = plsc.bitcast(magic - lax.shift_right_logical(bits, 1), jnp.float32)
    half_v = 0.5 * v
    for _ in range(3):
        y = y * (1.5 - half_v * y * y)
    return y


@functools.cache
def _build(B, S, H, s_chunk):
    nvec = H // LANES
    rows_per_w = B // NW
    n_schunk = S // s_chunk
    mesh = plsc.VectorSubcoreMesh(core_axis_name="c", subcore_axis_name="s")

    @functools.partial(
        pl.kernel,
        out_type=jax.ShapeDtypeStruct((B, S, H), jnp.float32),
        mesh=mesh,
        compiler_params=pltpu.CompilerParams(needs_layout_passes=False),
        scratch_types=[
            pltpu.VMEM((s_chunk,), jnp.int32),
            pltpu.VMEM((s_chunk, H), jnp.float32),
            pltpu.VMEM((s_chunk, H), jnp.float32),
            pltpu.VMEM((H,), jnp.float32),
            pltpu.VMEM((H,), jnp.float32),
            pltpu.SemaphoreType.DMA,
        ],
    )
    def bert_embed(tokens_hbm, word_hbm, pos_hbm, gamma_hbm, beta_hbm, out_hbm,
                   idx_v, rows_v, pos_v, gamma_v, beta_v, sem):
        wid = lax.axis_index("s") * NC + lax.axis_index("c")
        row0 = wid * rows_per_w
        pltpu.sync_copy(gamma_hbm, gamma_v)
        pltpu.sync_copy(beta_hbm, beta_v)

        def schunk_body(sc, _):
            s0 = pl.multiple_of(sc * s_chunk, s_chunk)
            pltpu.sync_copy(pos_hbm.at[pl.ds(s0, s_chunk)], pos_v)

            def batch_body(j, _):
                b = row0 + j
                pltpu.sync_copy(tokens_hbm.at[pl.ds(b * S + s0, s_chunk)], idx_v)
                pltpu.async_copy(word_hbm.at[idx_v], rows_v, sem).wait()

                def tok_body(t, _):
                    xs = [rows_v[t, pl.ds(i * LANES, LANES)]
                          + pos_v[t, pl.ds(i * LANES, LANES)]
                          for i in range(nvec)]
                    acc = [x for x in xs]
                    acc2 = [x * x for x in xs]
                    while len(acc) > 1:
                        acc = [acc[i] + acc[i + 1] for i in range(0, len(acc) - 1, 2)] \
                            + ([acc[-1]] if len(acc) % 2 else [])
                        acc2 = [acc2[i] + acc2[i + 1] for i in range(0, len(acc2) - 1, 2)] \
                            + ([acc2[-1]] if len(acc2) % 2 else [])
                    tot = jnp.sum(acc[0])
                    tot2 = jnp.sum(acc2[0])
                    mean = tot * (1.0 / H)
                    var = tot2 * (1.0 / H) - mean * mean
                    rinv = _rsqrt_vec(jnp.full((LANES,), var + EPS, dtype=jnp.float32))
                    mean_v = jnp.full((LANES,), mean, dtype=jnp.float32)
                    for i in range(nvec):
                        g = gamma_v[pl.ds(i * LANES, LANES)]
                        bt = beta_v[pl.ds(i * LANES, LANES)]
                        rows_v[t, pl.ds(i * LANES, LANES)] = \
                            (xs[i] - mean_v) * rinv * g + bt
                    return 0

                lax.fori_loop(0, s_chunk, tok_body, 0)
                pltpu.sync_copy(rows_v, out_hbm.at[b, pl.ds(s0, s_chunk)])
                return 0

            lax.fori_loop(0, rows_per_w, batch_body, 0)
            return 0

        lax.fori_loop(0, n_schunk, schunk_body, 0)

    return bert_embed


def kernel(tokens, word_table, pos_table, ln_gamma, ln_beta):
    B, S = tokens.shape
    H = word_table.shape[1]
    fn = _build(B, S, H, 40 if S % 40 == 0 else S)
    return fn(tokens.reshape(-1), word_table, pos_table[:S], ln_gamma, ln_beta)


# speedup vs baseline: 1.4184x; 1.4184x over previous
import functools

import jax
import jax.numpy as jnp
from jax import lax
from jax.experimental import pallas as pl
from jax.experimental.pallas import tpu as pltpu
from jax.experimental.pallas import tpu_sc as plsc

EPS = 1e-12
NC, NS, LANES = 2, 16, 16
NW = NC * NS


def _rsqrt_vec(v):
    bits = plsc.bitcast(v, jnp.int32)
    magic = jnp.full((LANES,), 0x5F3759DF, dtype=jnp.int32)
    y = plsc.bitcast(magic - lax.shift_right_logical(bits, 1), jnp.float32)
    half_v = 0.5 * v
    for _ in range(3):
        y = y * (1.5 - half_v * y * y)
    return y


@functools.cache
def _build(B, S, H, s_chunk):
    nvec = H // LANES
    rows_per_w = B // NW
    n_schunk = S // s_chunk
    mesh = plsc.VectorSubcoreMesh(core_axis_name="c", subcore_axis_name="s")

    @functools.partial(
        pl.kernel,
        out_type=jax.ShapeDtypeStruct((B, S, H), jnp.float32),
        mesh=mesh,
        compiler_params=pltpu.CompilerParams(needs_layout_passes=False),
        scratch_types=[
            pltpu.VMEM((s_chunk,), jnp.int32),
            pltpu.VMEM((s_chunk, H), jnp.float32),
            pltpu.VMEM((s_chunk, H), jnp.float32),
            pltpu.VMEM((H,), jnp.float32),
            pltpu.VMEM((H,), jnp.float32),
            pltpu.SemaphoreType.DMA,
        ],
    )
    def bert_embed(tokens_hbm, word_hbm, pos_hbm, gamma_hbm, beta_hbm, out_hbm,
                   idx_v, rows_v, pos_v, gamma_v, beta_v, sem):
        wid = lax.axis_index("s") * NC + lax.axis_index("c")
        row0 = wid * rows_per_w
        pltpu.sync_copy(gamma_hbm, gamma_v)
        pltpu.sync_copy(beta_hbm, beta_v)

        def schunk_body(sc, _):
            s0 = pl.multiple_of(sc * s_chunk, s_chunk)
            pltpu.sync_copy(pos_hbm.at[pl.ds(s0, s_chunk)], pos_v)

            def batch_body(j, _):
                b = row0 + j
                pltpu.sync_copy(tokens_hbm.at[pl.ds(b * S + s0, s_chunk)], idx_v)
                pltpu.async_copy(word_hbm.at[idx_v], rows_v, sem).wait()

                @plsc.parallel_loop(0, s_chunk, unroll=2)
                def tok_body(t):
                    xs = [rows_v[t, pl.ds(i * LANES, LANES)]
                          + pos_v[t, pl.ds(i * LANES, LANES)]
                          for i in range(nvec)]
                    acc = [x for x in xs]
                    acc2 = [x * x for x in xs]
                    while len(acc) > 1:
                        acc = [acc[i] + acc[i + 1] for i in range(0, len(acc) - 1, 2)] \
                            + ([acc[-1]] if len(acc) % 2 else [])
                        acc2 = [acc2[i] + acc2[i + 1] for i in range(0, len(acc2) - 1, 2)] \
                            + ([acc2[-1]] if len(acc2) % 2 else [])
                    tot = jnp.sum(acc[0])
                    tot2 = jnp.sum(acc2[0])
                    mean = tot * (1.0 / H)
                    var = tot2 * (1.0 / H) - mean * mean
                    rinv = _rsqrt_vec(jnp.full((LANES,), var + EPS, dtype=jnp.float32))
                    mean_v = jnp.full((LANES,), mean, dtype=jnp.float32)
                    for i in range(nvec):
                        g = gamma_v[pl.ds(i * LANES, LANES)]
                        bt = beta_v[pl.ds(i * LANES, LANES)]
                        rows_v[t, pl.ds(i * LANES, LANES)] = \
                            (xs[i] - mean_v) * rinv * g + bt

                pltpu.sync_copy(rows_v, out_hbm.at[b, pl.ds(s0, s_chunk)])
                return 0

            lax.fori_loop(0, rows_per_w, batch_body, 0)
            return 0

        lax.fori_loop(0, n_schunk, schunk_body, 0)

    return bert_embed


def kernel(tokens, word_table, pos_table, ln_gamma, ln_beta):
    B, S = tokens.shape
    H = word_table.shape[1]
    fn = _build(B, S, H, 40 if S % 40 == 0 else S)
    return fn(tokens.reshape(-1), word_table, pos_table[:S], ln_gamma, ln_beta)
